# baseline (speedup 1.0000x reference)
import functools

import jax
import jax.numpy as jnp
from jax import lax
from jax.experimental import pallas as pl
from jax.experimental.pallas import tpu as pltpu
from jax.experimental.pallas import tpu_sc as plsc

N_NODES = 10000
N_EDGES = 320000
D = 128
LANES = 16

NC, NS = 2, 16
NW = NC * NS
E_PER_W = N_EDGES // NW
CH = 104
NCHUNK = 96
TAIL = E_PER_W - NCHUNK * CH
NG = 3
NO = 2
STEP = 6

_mesh = plsc.VectorSubcoreMesh(core_axis_name="c", subcore_axis_name="s")


@functools.partial(
    pl.kernel,
    out_type=jax.ShapeDtypeStruct((N_EDGES, D), jnp.float32),
    mesh=_mesh,
    scratch_types=[
        pltpu.VMEM((E_PER_W,), jnp.int32),
        pltpu.VMEM((E_PER_W,), jnp.int32),
        pltpu.VMEM((NG, CH, D), jnp.float32),
        pltpu.VMEM((NG, CH, D), jnp.float32),
        pltpu.VMEM((NO, CH, D), jnp.float32),
        pltpu.SemaphoreType.DMA,
        pltpu.SemaphoreType.DMA,
        pltpu.SemaphoreType.DMA,
        pltpu.SemaphoreType.DMA,
        pltpu.SemaphoreType.DMA,
        pltpu.SemaphoreType.DMA,
        pltpu.SemaphoreType.DMA,
        pltpu.SemaphoreType.DMA,
    ],
)
def _ratios_sc(table, idx_i, idx_j, out, ii_v, jj_v, ri_v, rj_v, ro_v,
               sgi0, sgi1, sgi2, sgj0, sgj1, sgj2, so0, so1):
    wid = lax.axis_index("s") * NC + lax.axis_index("c")
    base = wid * E_PER_W
    pltpu.sync_copy(idx_i.at[pl.ds(base, E_PER_W)], ii_v)
    pltpu.sync_copy(idx_j.at[pl.ds(base, E_PER_W)], jj_v)
    sgi = (sgi0, sgi1, sgi2)
    sgj = (sgj0, sgj1, sgj2)
    so = (so0, so1)

    toff = NCHUNK * CH
    pltpu.sync_copy(table.at[ii_v.at[pl.ds(toff, TAIL)]], ri_v.at[0, pl.ds(0, TAIL)])
    pltpu.sync_copy(table.at[jj_v.at[pl.ds(toff, TAIL)]], rj_v.at[0, pl.ds(0, TAIL)])

    def tail_body(r, rcarry):
        for k in range(D // LANES):
            s = pl.ds(k * LANES, LANES)
            ro_v[0, r, s] = ri_v[0, r, s] - rj_v[0, r, s]
        return rcarry

    lax.fori_loop(0, TAIL, tail_body, 0, unroll=4)
    pltpu.sync_copy(ro_v.at[0, pl.ds(0, TAIL)], out.at[pl.ds(base + toff, TAIL)])

    def issue_gathers(c, g):
        off = c * CH
        pltpu.async_copy(table.at[ii_v.at[pl.ds(off, CH)]], ri_v.at[g], sgi[g])
        pltpu.async_copy(table.at[jj_v.at[pl.ds(off, CH)]], rj_v.at[g], sgj[g])

    issue_gathers(0, 0)
    issue_gathers(1, 1)
    issue_gathers(2, 2)

    def iter_body(it, carry):
        for u in range(STEP):
            c = it * STEP + u
            g = u % NG
            o = u % NO
            pltpu.make_async_copy(table.at[ii_v.at[pl.ds(0, CH)]], ri_v.at[g], sgi[g]).wait()
            pltpu.make_async_copy(table.at[jj_v.at[pl.ds(0, CH)]], rj_v.at[g], sgj[g]).wait()
            @pl.when(c >= NO)
            def _():
                pltpu.make_async_copy(ro_v.at[o], out.at[pl.ds(0, CH)], so[o]).wait()

            def row_body(r, rcarry):
                for k in range(D // LANES):
                    s = pl.ds(k * LANES, LANES)
                    ro_v[o, r, s] = ri_v[g, r, s] - rj_v[g, r, s]
                return rcarry

            lax.fori_loop(0, CH, row_body, 0, unroll=4)

            @pl.when(c + NG < NCHUNK)
            def _():
                issue_gathers(c + NG, g)

            pltpu.async_copy(ro_v.at[o], out.at[pl.ds(base + c * CH, CH)], so[o])
        return carry

    lax.fori_loop(0, NCHUNK // STEP, iter_body, 0, unroll=False)
    pltpu.make_async_copy(ro_v.at[0], out.at[pl.ds(0, CH)], so[0]).wait()
    pltpu.make_async_copy(ro_v.at[1], out.at[pl.ds(0, CH)], so[1]).wait()


def _weights_tc_body(w_ref, o_ref):
    o_ref[...] = jnp.broadcast_to(w_ref[...], o_ref.shape)


_W_BLK = 3200


def _weights_tc(weights):
    return pl.pallas_call(
        _weights_tc_body,
        grid=(N_EDGES // _W_BLK,),
        in_specs=[pl.BlockSpec((1, D), lambda i: (0, 0))],
        out_specs=pl.BlockSpec((_W_BLK, D), lambda i: (i, 0)),
        out_shape=jax.ShapeDtypeStruct((N_EDGES, D), jnp.float32),
    )(weights)


def kernel(particles, weights, edges):
    table = particles.reshape(N_NODES, D)
    idx = edges.astype(jnp.int32)
    idx_i = idx[:, 0]
    idx_j = idx[:, 1]
    ratios = _ratios_sc(table, idx_i, idx_j)
    rm_weights = _weights_tc(weights)
    return ratios.reshape(N_EDGES, D, 1), rm_weights
